# baseline (speedup 1.0000x reference)
import functools

import jax
import jax.numpy as jnp
from jax import lax
from jax.experimental import pallas as pl
from jax.experimental.pallas import tpu as pltpu
from jax.experimental.pallas import tpu_sc as plsc

_NUM_HOPS = 3
_SIGMA = 0.1
_N = 10000
_D = 128
_E = 320000

_NC = 2
_NS = 16
_NW = _NC * _NS
_EPW = _E // _NW
_C = 40
_NCHUNK = _EPW // _C
_NP = 10240
_RPS = _NP // _NS
_RC = _C


def _sc_hop_body(h_hbm, src_hbm, dst_hbm, out_hbm,
                 aggr_sh, sidx_all, didx_all, srows0, srows1, drows0, drows1,
                 didx_c0, didx_c1, tbuf, gsem0, gsem1, ssem0, ssem1):
    c = lax.axis_index("c")
    s = lax.axis_index("s")
    wid = c * _NS + s

    z16 = jnp.zeros((16,), jnp.float32)
    lanes = lax.iota(jnp.int32, 16)

    ebase = wid * _EPW
    cp_si = pltpu.async_copy(src_hbm.at[pl.ds(ebase, _EPW)], sidx_all, gsem0)
    cp_di = pltpu.async_copy(dst_hbm.at[pl.ds(ebase, _EPW)], didx_all, gsem0)

    def zero_row(i, _):
        for g in range(_D // 16):
            srows0[i, pl.ds(g * 16, 16)] = z16
        return 0

    lax.fori_loop(0, _C, zero_row, 0)
    for j in range(_RPS // _RC):
        pltpu.sync_copy(srows0, aggr_sh.at[pl.ds(s * _RPS + j * _RC, _RC)])
    cp_si.wait()
    cp_di.wait()
    plsc.subcore_barrier()

    def fire(ci, sr, dr, gs):
        sl = pl.ds(ci * _C, _C)
        pltpu.async_copy(h_hbm.at[sidx_all.at[sl]], sr, gs)
        pltpu.async_copy(h_hbm.at[didx_all.at[sl]], dr, gs)

    def wait_gathers(ci, sr, dr, gs):
        sl = pl.ds(ci * _C, _C)
        pltpu.make_async_copy(h_hbm.at[sidx_all.at[sl]], sr, gs).wait()
        pltpu.make_async_copy(h_hbm.at[didx_all.at[sl]], dr, gs).wait()

    def stage_didx(ci, dc):
        for off in (0, 16, _C - 16):
            dc[pl.ds(off, 16)] = didx_all[pl.ds(ci * _C + off, 16)]

    def fire_scatter(sr, dc, ss):
        pltpu.async_copy(sr, aggr_sh.at[dc], ss, add=True)

    def wait_scatter(sr, dc, ss):
        pltpu.make_async_copy(sr, aggr_sh.at[dc], ss).wait()

    def emit_group(sr, dr, base_row, nrows):
        for e in range(nrows):
            row = base_row + e
            acc = sr[row, pl.ds(0, 16)] * dr[row, pl.ds(0, 16)]
            for k in range(1, _D // 16):
                sl = pl.ds(k * 16, 16)
                acc = acc + sr[row, sl] * dr[row, sl]
            tbuf[pl.ds(e * 17, 16)] = acc
        tot = plsc.load_gather(tbuf, [lanes * 17])
        for col in range(1, 16):
            tot = tot + plsc.load_gather(tbuf, [lanes * 17 + col])
        alpha = 1.0 / (1.0 + jnp.exp(-tot))
        for e in range(nrows):
            row = base_row + e
            a = alpha[e]
            for k in range(_D // 16):
                sl = pl.ds(k * 16, 16)
                sr[row, sl] = sr[row, sl] * a

    def compute(sr, dr):
        def group_body(g, _):
            emit_group(sr, dr, g * 16, 16)
            return 0

        lax.fori_loop(0, _C // 16, group_body, 0)
        if _C % 16:
            emit_group(sr, dr, (_C // 16) * 16, _C % 16)

    def zero_srow1(i, _):
        for g in range(_D // 16):
            srows1[i, pl.ds(g * 16, 16)] = z16
        return 0

    lax.fori_loop(0, _C, zero_srow1, 0)
    stage_didx(0, didx_c1)
    fire_scatter(srows1, didx_c1, ssem1)
    fire(0, srows0, drows0, gsem0)

    def pair_body(gi, _):
        ci = 2 * gi
        wait_gathers(ci, srows0, drows0, gsem0)
        wait_scatter(srows1, didx_c1, ssem1)
        fire(ci + 1, srows1, drows1, gsem1)
        compute(srows0, drows0)
        stage_didx(ci, didx_c0)
        fire_scatter(srows0, didx_c0, ssem0)
        wait_gathers(ci + 1, srows1, drows1, gsem1)
        wait_scatter(srows0, didx_c0, ssem0)
        fire(ci + 2, srows0, drows0, gsem0)
        compute(srows1, drows1)
        stage_didx(ci + 1, didx_c1)
        fire_scatter(srows1, didx_c1, ssem1)
        return 0

    lax.fori_loop(0, _NCHUNK // 2 - 1, pair_body, 0)
    ci = _NCHUNK - 2
    wait_gathers(ci, srows0, drows0, gsem0)
    wait_scatter(srows1, didx_c1, ssem1)
    fire(ci + 1, srows1, drows1, gsem1)
    compute(srows0, drows0)
    stage_didx(ci, didx_c0)
    fire_scatter(srows0, didx_c0, ssem0)
    wait_gathers(ci + 1, srows1, drows1, gsem1)
    wait_scatter(srows0, didx_c0, ssem0)
    compute(srows1, drows1)
    stage_didx(ci + 1, didx_c1)
    fire_scatter(srows1, didx_c1, ssem1)
    wait_scatter(srows1, didx_c1, ssem1)
    plsc.subcore_barrier()

    for j in range(_RPS // _RC):
        rb = s * _RPS + j * _RC
        pltpu.sync_copy(aggr_sh.at[pl.ds(rb, _RC)], srows0)
        pltpu.sync_copy(srows0, out_hbm.at[c, pl.ds(rb, _RC)])


_sc_hop = functools.partial(
    pl.kernel,
    out_type=jax.ShapeDtypeStruct((_NC, _NP, _D), jnp.float32),
    mesh=plsc.VectorSubcoreMesh(
        core_axis_name="c", subcore_axis_name="s",
        num_cores=_NC, num_subcores=_NS),
    compiler_params=pltpu.CompilerParams(needs_layout_passes=False),
    scratch_types=[
        pltpu.VMEM_SHARED((_NP, _D), jnp.float32),
        pltpu.VMEM((_EPW,), jnp.int32),
        pltpu.VMEM((_EPW,), jnp.int32),
        pltpu.VMEM((_C, _D), jnp.float32),
        pltpu.VMEM((_C, _D), jnp.float32),
        pltpu.VMEM((_C, _D), jnp.float32),
        pltpu.VMEM((_C, _D), jnp.float32),
        pltpu.VMEM((_C,), jnp.int32),
        pltpu.VMEM((_C,), jnp.int32),
        pltpu.VMEM((16 * 17,), jnp.float32),
        pltpu.SemaphoreType.DMA,
        pltpu.SemaphoreType.DMA,
        pltpu.SemaphoreType.DMA,
        pltpu.SemaphoreType.DMA,
    ],
)(_sc_hop_body)


def _normalize_rows(y):
    ss = jnp.sum(y * y, axis=1, keepdims=True)
    return y / jnp.maximum(jnp.sqrt(ss), 1e-12)


def _tc_normalize_body(x_ref, o_ref):
    o_ref[...] = _normalize_rows(x_ref[...])


def _tc_combine_body(p0_ref, p1_ref, nz_ref, o_ref):
    o_ref[...] = _normalize_rows(p0_ref[...] + p1_ref[...] + nz_ref[...])


_TC_BLK = 1000

_tc_normalize = pl.pallas_call(
    _tc_normalize_body,
    grid=(_N // _TC_BLK,),
    in_specs=[pl.BlockSpec((_TC_BLK, _D), lambda i: (i, 0))],
    out_specs=pl.BlockSpec((_TC_BLK, _D), lambda i: (i, 0)),
    out_shape=jax.ShapeDtypeStruct((_N, _D), jnp.float32),
)

_tc_combine = pl.pallas_call(
    _tc_combine_body,
    grid=(_N // _TC_BLK,),
    in_specs=[pl.BlockSpec((_TC_BLK, _D), lambda i: (i, 0))] * 3,
    out_specs=pl.BlockSpec((_TC_BLK, _D), lambda i: (i, 0)),
    out_shape=jax.ShapeDtypeStruct((_N, _D), jnp.float32),
)


def kernel(x, edge_index):
    src = edge_index[0]
    dst = edge_index[1]
    h = _tc_normalize(x)
    outs = [h]
    for k in range(_NUM_HOPS):
        noise = _SIGMA * jax.random.normal(
            jax.random.fold_in(jax.random.key(1), k), (_N, _D),
            dtype=jnp.float32)
        parts = _sc_hop(h, src, dst)
        h = _tc_combine(parts[0, :_N], parts[1, :_N], noise)
        outs.append(h)
    return jnp.stack(outs)


# speedup vs baseline: 1.4751x; 1.4751x over previous
import functools

import jax
import jax.numpy as jnp
from jax import lax
from jax.experimental import pallas as pl
from jax.experimental.pallas import tpu as pltpu
from jax.experimental.pallas import tpu_sc as plsc

_NUM_HOPS = 3
_SIGMA = 0.1
_N = 10000
_D = 128
_E = 320000

_NC = 2
_NS = 16
_NW = _NC * _NS
_EPW = _E // _NW
_C = 80
_NCHUNK = _EPW // _C
_NP = 10240
_RPS = _NP // _NS
_RC = _C


def _sc_hop_body(h_hbm, src_hbm, dst_hbm, out_hbm,
                 aggr_sh, sidx0, sidx1, didx0, didx1,
                 srows0, srows1, drows0, drows1, tbuf,
                 gsem0, gsem1, ssem0, ssem1):
    c = lax.axis_index("c")
    s = lax.axis_index("s")
    wid = c * _NS + s
    ebase = wid * _EPW

    z16 = jnp.zeros((16,), jnp.float32)
    lanes = lax.iota(jnp.int32, 16)

    def zero_rows(buf):
        def zero_row(i, _):
            for g in range(_D // 16):
                buf[i, pl.ds(g * 16, 16)] = z16
            return 0

        lax.fori_loop(0, _C, zero_row, 0)

    zero_rows(srows0)
    for j in range(_RPS // _RC):
        pltpu.sync_copy(srows0, aggr_sh.at[pl.ds(s * _RPS + j * _RC, _RC)])
    plsc.subcore_barrier()

    def fire(ci, si, di, sr, dr, gs):
        base = ebase + ci * _C
        pltpu.sync_copy(src_hbm.at[pl.ds(base, _C)], si)
        pltpu.sync_copy(dst_hbm.at[pl.ds(base, _C)], di)
        pltpu.async_copy(h_hbm.at[si], sr, gs)
        pltpu.async_copy(h_hbm.at[di], dr, gs)

    def wait_gathers(si, di, sr, dr, gs):
        pltpu.make_async_copy(h_hbm.at[si], sr, gs).wait()
        pltpu.make_async_copy(h_hbm.at[di], dr, gs).wait()

    def fire_scatter(sr, di, ss):
        pltpu.async_copy(sr, aggr_sh.at[di], ss, add=True)

    def wait_scatter(sr, di, ss):
        pltpu.make_async_copy(sr, aggr_sh.at[di], ss).wait()

    def emit_group(sr, dr, base_row):
        for e in range(16):
            row = base_row + e
            acc = sr[row, pl.ds(0, 16)] * dr[row, pl.ds(0, 16)]
            for k in range(1, _D // 16):
                sl = pl.ds(k * 16, 16)
                acc = acc + sr[row, sl] * dr[row, sl]
            tbuf[pl.ds(e * 17, 16)] = acc
        tot = plsc.load_gather(tbuf, [lanes * 17])
        for col in range(1, 16):
            tot = tot + plsc.load_gather(tbuf, [lanes * 17 + col])
        alpha = 1.0 / (1.0 + jnp.exp(-tot))
        for e in range(16):
            row = base_row + e
            a = alpha[e]
            for k in range(_D // 16):
                sl = pl.ds(k * 16, 16)
                sr[row, sl] = sr[row, sl] * a

    def compute(sr, dr):
        def group_body(g, _):
            emit_group(sr, dr, g * 16)
            return 0

        lax.fori_loop(0, _C // 16, group_body, 0)

    zero_rows(srows1)
    pltpu.sync_copy(dst_hbm.at[pl.ds(ebase, _C)], didx1)
    fire_scatter(srows1, didx1, ssem1)
    fire(0, sidx0, didx0, srows0, drows0, gsem0)

    def pair_body(gi, _):
        ci = 2 * gi
        wait_gathers(sidx0, didx0, srows0, drows0, gsem0)
        wait_scatter(srows1, didx1, ssem1)
        fire(ci + 1, sidx1, didx1, srows1, drows1, gsem1)
        compute(srows0, drows0)
        fire_scatter(srows0, didx0, ssem0)
        wait_gathers(sidx1, didx1, srows1, drows1, gsem1)
        wait_scatter(srows0, didx0, ssem0)
        fire(ci + 2, sidx0, didx0, srows0, drows0, gsem0)
        compute(srows1, drows1)
        fire_scatter(srows1, didx1, ssem1)
        return 0

    lax.fori_loop(0, (_NCHUNK - 1) // 2, pair_body, 0)
    wait_gathers(sidx0, didx0, srows0, drows0, gsem0)
    wait_scatter(srows1, didx1, ssem1)
    compute(srows0, drows0)
    fire_scatter(srows0, didx0, ssem0)
    wait_scatter(srows0, didx0, ssem0)
    plsc.subcore_barrier()

    for j in range(_RPS // _RC):
        rb = s * _RPS + j * _RC
        pltpu.sync_copy(aggr_sh.at[pl.ds(rb, _RC)], srows0)
        pltpu.sync_copy(srows0, out_hbm.at[c, pl.ds(rb, _RC)])


_sc_hop = functools.partial(
    pl.kernel,
    out_type=jax.ShapeDtypeStruct((_NC, _NP, _D), jnp.float32),
    mesh=plsc.VectorSubcoreMesh(
        core_axis_name="c", subcore_axis_name="s",
        num_cores=_NC, num_subcores=_NS),
    compiler_params=pltpu.CompilerParams(needs_layout_passes=False),
    scratch_types=[
        pltpu.VMEM_SHARED((_NP, _D), jnp.float32),
        pltpu.VMEM((_C,), jnp.int32),
        pltpu.VMEM((_C,), jnp.int32),
        pltpu.VMEM((_C,), jnp.int32),
        pltpu.VMEM((_C,), jnp.int32),
        pltpu.VMEM((_C, _D), jnp.float32),
        pltpu.VMEM((_C, _D), jnp.float32),
        pltpu.VMEM((_C, _D), jnp.float32),
        pltpu.VMEM((_C, _D), jnp.float32),
        pltpu.VMEM((16 * 17,), jnp.float32),
        pltpu.SemaphoreType.DMA,
        pltpu.SemaphoreType.DMA,
        pltpu.SemaphoreType.DMA,
        pltpu.SemaphoreType.DMA,
    ],
)(_sc_hop_body)


def _normalize_rows(y):
    ss = jnp.sum(y * y, axis=1, keepdims=True)
    return y / jnp.maximum(jnp.sqrt(ss), 1e-12)


def _tc_normalize_body(x_ref, o_ref):
    o_ref[...] = _normalize_rows(x_ref[...])


def _tc_combine_body(p0_ref, p1_ref, nz_ref, o_ref):
    o_ref[...] = _normalize_rows(p0_ref[...] + p1_ref[...] + nz_ref[...])


_TC_BLK = 1000

_tc_normalize = pl.pallas_call(
    _tc_normalize_body,
    grid=(_N // _TC_BLK,),
    in_specs=[pl.BlockSpec((_TC_BLK, _D), lambda i: (i, 0))],
    out_specs=pl.BlockSpec((_TC_BLK, _D), lambda i: (i, 0)),
    out_shape=jax.ShapeDtypeStruct((_N, _D), jnp.float32),
)

_tc_combine = pl.pallas_call(
    _tc_combine_body,
    grid=(_N // _TC_BLK,),
    in_specs=[pl.BlockSpec((_TC_BLK, _D), lambda i: (i, 0))] * 3,
    out_specs=pl.BlockSpec((_TC_BLK, _D), lambda i: (i, 0)),
    out_shape=jax.ShapeDtypeStruct((_N, _D), jnp.float32),
)


def kernel(x, edge_index):
    src = edge_index[0]
    dst = edge_index[1]
    h = _tc_normalize(x)
    outs = [h]
    for k in range(_NUM_HOPS):
        noise = _SIGMA * jax.random.normal(
            jax.random.fold_in(jax.random.key(1), k), (_N, _D),
            dtype=jnp.float32)
        parts = _sc_hop(h, src, dst)
        h = _tc_combine(parts[0, :_N], parts[1, :_N], noise)
        outs.append(h)
    return jnp.stack(outs)
